# baseline (speedup 1.0000x reference)
import functools

import jax
import jax.numpy as jnp
from jax import lax
from jax.experimental import pallas as pl
from jax.experimental.pallas import tpu as pltpu
from jax.experimental.pallas import tpu_sc as plsc

_NC = 2
_NS = 16
_C = 64


def _sc_replicate(xf, r):
    rows, d = xf.shape
    nw = _NC * _NS
    rpw = rows // nw
    mesh = plsc.VectorSubcoreMesh(core_axis_name="c", subcore_axis_name="s")

    @functools.partial(
        pl.kernel,
        out_type=jax.ShapeDtypeStruct((rows, r, d), xf.dtype),
        mesh=mesh,
        scratch_types=[pltpu.SemaphoreType.DMA],
    )
    def run(x_hbm, out_hbm, sem):
        wid = lax.axis_index("s") * _NC + lax.axis_index("c")
        base = wid * rpw
        for j in range(r):
            pltpu.make_async_copy(
                x_hbm.at[pl.ds(base, rpw), :],
                out_hbm.at[pl.ds(base, rpw), j, :],
                sem,
            ).start()
        for j in range(r):
            pltpu.make_async_copy(
                x_hbm.at[pl.ds(base, rpw), :],
                out_hbm.at[pl.ds(base, rpw), j, :],
                sem,
            ).wait()

    return run(xf)


def kernel(x, idx):
    b, t, d = x.shape
    r = idx.shape[1]
    out = _sc_replicate(x.reshape(b * t, d), r)
    out = out.reshape(b, t, r, d)
    return jnp.expand_dims(jnp.transpose(out, (0, 1, 3, 2)), 3)


# speedup vs baseline: 52.2746x; 52.2746x over previous
import functools

import jax
import jax.numpy as jnp
from jax import lax
from jax.experimental import pallas as pl
from jax.experimental.pallas import tpu as pltpu
from jax.experimental.pallas import tpu_sc as plsc

_NC = 2
_NS = 16
_C = 64


def _sc_replicate(xf, r):
    rows, d = xf.shape
    nw = _NC * _NS
    rpw = rows // nw
    mesh = plsc.VectorSubcoreMesh(core_axis_name="c", subcore_axis_name="s")

    nchunk = rpw // _C

    @functools.partial(
        pl.kernel,
        out_type=jax.ShapeDtypeStruct((rows, r, d), xf.dtype),
        mesh=mesh,
        scratch_types=[
            pltpu.VMEM((_C, d), xf.dtype),
            pltpu.SemaphoreType.DMA,
            pltpu.SemaphoreType.DMA,
        ],
    )
    def run(x_hbm, out_hbm, buf, sem_in, sem_out):
        wid = lax.axis_index("s") * _NC + lax.axis_index("c")
        base = wid * rpw

        def body(i, carry):
            s = base + i * _C
            cp = pltpu.make_async_copy(x_hbm.at[pl.ds(s, _C), :], buf, sem_in)
            cp.start()
            cp.wait()
            for j in range(r):
                pltpu.make_async_copy(
                    buf, out_hbm.at[pl.ds(s, _C), j, :], sem_out
                ).start()
            for j in range(r):
                pltpu.make_async_copy(
                    buf, out_hbm.at[pl.ds(s, _C), j, :], sem_out
                ).wait()
            return carry

        lax.fori_loop(0, nchunk, body, 0)

    return run(xf)


def kernel(x, idx):
    b, t, d = x.shape
    r = idx.shape[1]
    out = _sc_replicate(x.reshape(b * t, d), r)
    out = out.reshape(b, t, r, d)
    return jnp.expand_dims(jnp.transpose(out, (0, 1, 3, 2)), 3)
